# baseline (speedup 1.0000x reference)
import functools

import jax
import jax.numpy as jnp
from jax import lax
from jax.experimental import pallas as pl
from jax.experimental.pallas import tpu as pltpu
from jax.experimental.pallas import tpu_sc as plsc

NC, NS, L = 2, 16, 16
NW = NC * NS
B = 16384
BS = B // 2
CHUNK = B // NW
NBLK = CHUNK // L
PAD = 512
OUT_ROWS = BS + PAD
D = 128

_mesh = plsc.VectorSubcoreMesh(core_axis_name="c", subcore_axis_name="s",
                               num_cores=NC, num_subcores=NS)


def _wid():
    return lax.axis_index("s") * NC + lax.axis_index("c")


@functools.partial(
    pl.kernel,
    out_type=jax.ShapeDtypeStruct((NW, L), jnp.int32),
    mesh=_mesh,
    scratch_types=[
        pltpu.VMEM((CHUNK,), jnp.int32),
        pltpu.VMEM((CHUNK,), jnp.int32),
        pltpu.VMEM((L,), jnp.int32),
    ],
    compiler_params=pltpu.CompilerParams(needs_layout_passes=False),
)
def _count_kernel(fav_hbm, ret_hbm, cnt_hbm, fav_v, ret_v, out_v):
    w = _wid()
    base = w * CHUNK
    pltpu.sync_copy(fav_hbm.at[pl.ds(base, CHUNK)], fav_v)
    pltpu.sync_copy(ret_hbm.at[pl.ds(base, CHUNK)], ret_v)
    cnt = jnp.zeros((L,), jnp.int32)
    for b in range(NBLK):
        f = fav_v[pl.ds(b * L, L)]
        r = ret_v[pl.ds(b * L, L)]
        m = (f == 1) | (r == 1)
        cnt = cnt + plsc.all_reduce_population_count(m)
    out_v[...] = cnt
    pltpu.sync_copy(out_v, cnt_hbm.at[w])


@functools.partial(
    pl.kernel,
    out_type=(
        jax.ShapeDtypeStruct((OUT_ROWS * 4,), jnp.int32),
        jax.ShapeDtypeStruct((OUT_ROWS, D), jnp.float32),
    ),
    mesh=_mesh,
    scratch_types=[
        pltpu.VMEM((CHUNK,), jnp.int32),
        pltpu.VMEM((CHUNK,), jnp.int32),
        pltpu.VMEM((2 * CHUNK,), jnp.float32),
        pltpu.VMEM((NW, L), jnp.int32),
        pltpu.VMEM((CHUNK, D), jnp.float32),
        pltpu.VMEM((4 * CHUNK,), jnp.int32),
        pltpu.VMEM((4, CHUNK // 4), jnp.int32),
        pltpu.VMEM((16, CHUNK // 4), jnp.int32),
        pltpu.SemaphoreType.DMA,
        pltpu.SemaphoreType.DMA,
    ],
    compiler_params=pltpu.CompilerParams(needs_layout_passes=False,
                                         use_tc_tiling_on_sc=False),
)
def _partition_kernel(fav_hbm, ret_hbm, wfl_hbm, emb_hbm, cnt_hbm,
                      packed_out, emb_out,
                      fav_v, ret_v, w_v, cnt_v, emb_v, packed_v, dest_v,
                      dest4_v, ldsem, stsem):
    w = _wid()
    base = w * CHUNK
    emb_cp = pltpu.async_copy(emb_hbm.at[pl.ds(base, CHUNK), :], emb_v, ldsem)
    pltpu.sync_copy(fav_hbm.at[pl.ds(base, CHUNK)], fav_v)
    pltpu.sync_copy(ret_hbm.at[pl.ds(base, CHUNK)], ret_v)
    pltpu.sync_copy(wfl_hbm.at[pl.ds(2 * base, 2 * CHUNK)], w_v)
    pltpu.sync_copy(cnt_hbm, cnt_v)

    widx = lax.iota(jnp.int32, L)
    zero = jnp.zeros((L,), jnp.int32)
    c0 = plsc.load_gather(cnt_v, [widx, zero])
    c1 = plsc.load_gather(cnt_v, [widx + L, zero])
    npos = jnp.sum(c0) + jnp.sum(c1)
    pos_base = (jnp.sum(jnp.where(widx < w, c0, 0))
                + jnp.sum(jnp.where(widx + L < w, c1, 0)))
    neg_base = npos + base - pos_base

    npos_s = jnp.full((L,), npos, jnp.int32).astype(jnp.float32)
    den_s = jnp.float32(BS) - npos_s
    nw_s = jnp.where(den_s == 0.0, jnp.float32(0.0),
                     (jnp.float32(2 * BS) - npos_s) / den_s)
    pos_base_s = jnp.full((L,), pos_base, jnp.int32)
    neg_base_s = jnp.full((L,), neg_base, jnp.int32)
    lane = lax.iota(jnp.int32, L)
    ones = jnp.ones((L,), jnp.float32)
    cols = [jnp.full((L,), c, jnp.int32) for c in range(4)]
    carry_p = jnp.zeros((L,), jnp.int32)
    for b in range(NBLK):
        f = fav_v[pl.ds(b * L, L)]
        r = ret_v[pl.ds(b * L, L)]
        m = (f == 1) | (r == 1)
        mi = m.astype(jnp.int32)
        excl = plsc.cumsum(mi) - mi
        posd = pos_base_s + carry_p + excl
        negd = neg_base_s + (b * L - carry_p) + (lane - excl)
        dest = jnp.where(m, posd, negd)
        dest = jnp.where(dest >= BS, BS + (dest & (PAD - 1)), dest)
        dest_v[b // 8, pl.ds((b % 8) * L, L)] = dest
        carry_p = carry_p + plsc.all_reduce_population_count(m)
        rows = lane + (b * L)
        w0 = plsc.load_gather(w_v, [rows * 2])
        w1 = plsc.load_gather(w_v, [rows * 2 + 1])
        factor = jnp.where(m, ones, nw_s)
        p0 = lane * 4 + (b * 4 * L)
        vals = (f, r, plsc.bitcast(w0 * factor, jnp.int32),
                plsc.bitcast(w1 * factor, jnp.int32))
        for c in range(4):
            plsc.store_scatter(packed_v, [p0 + c], vals[c])
        d0 = dest * 4
        for c in range(4):
            p = p0 + c
            plsc.store_scatter(dest4_v, [jnp.right_shift(p, 7), p & 127], d0 + c)

    emb_cp.wait()
    copies = []
    for j in range(4):
        copies.append(
            pltpu.async_copy(emb_v.at[pl.ds(j * (CHUNK // 4), CHUNK // 4), :],
                             emb_out.at[dest_v.at[j]], stsem))
    for j in range(16):
        copies.append(
            pltpu.async_copy(packed_v.at[pl.ds(j * (CHUNK // 4), CHUNK // 4)],
                             packed_out.at[dest4_v.at[j]], stsem))
    for c in copies:
        c.wait()


def kernel(fav, retweet, embedding, weights):
    fav1 = fav.reshape(B)
    ret1 = retweet.reshape(B)
    wfl = weights.reshape(2 * B)
    cnts = _count_kernel(fav1, ret1)
    packed_flat, embp = _partition_kernel(fav1, ret1, wfl, embedding, cnts)
    packed = packed_flat.reshape(OUT_ROWS, 4)
    out_fav = packed[:BS, 0:1]
    out_ret = packed[:BS, 1:2]
    out_w = lax.bitcast_convert_type(packed[:BS, 2:4], jnp.float32)
    return out_fav, out_ret, embp[:BS], out_w


# speedup vs baseline: 7.8641x; 7.8641x over previous
import functools

import jax
import jax.numpy as jnp
from jax import lax
from jax.experimental import pallas as pl
from jax.experimental.pallas import tpu as pltpu
from jax.experimental.pallas import tpu_sc as plsc

NC, NS, L = 2, 16, 16
NW = NC * NS
B = 16384
BS = B // 2
CHUNK = B // NW
NBLK = CHUNK // L
PAD = 512
OUT_ROWS = BS + PAD
D = 128

_mesh = plsc.VectorSubcoreMesh(core_axis_name="c", subcore_axis_name="s",
                               num_cores=NC, num_subcores=NS)


def _wid():
    return lax.axis_index("s") * NC + lax.axis_index("c")


@functools.partial(
    pl.kernel,
    out_type=jax.ShapeDtypeStruct((NW, L), jnp.int32),
    mesh=_mesh,
    scratch_types=[
        pltpu.VMEM((CHUNK,), jnp.int32),
        pltpu.VMEM((CHUNK,), jnp.int32),
        pltpu.VMEM((L,), jnp.int32),
    ],
    compiler_params=pltpu.CompilerParams(needs_layout_passes=False),
)
def _count_kernel(fav_hbm, ret_hbm, cnt_hbm, fav_v, ret_v, out_v):
    w = _wid()
    base = w * CHUNK
    pltpu.sync_copy(fav_hbm.at[pl.ds(base, CHUNK)], fav_v)
    pltpu.sync_copy(ret_hbm.at[pl.ds(base, CHUNK)], ret_v)
    cnt = jnp.zeros((L,), jnp.int32)
    for b in range(NBLK):
        f = fav_v[pl.ds(b * L, L)]
        r = ret_v[pl.ds(b * L, L)]
        m = (f == 1) | (r == 1)
        cnt = cnt + plsc.all_reduce_population_count(m)
    out_v[...] = cnt
    pltpu.sync_copy(out_v, cnt_hbm.at[w])


@functools.partial(
    pl.kernel,
    out_type=(
        jax.ShapeDtypeStruct((OUT_ROWS, D), jnp.int32),
        jax.ShapeDtypeStruct((OUT_ROWS, D), jnp.float32),
    ),
    mesh=_mesh,
    scratch_types=[
        pltpu.VMEM((CHUNK,), jnp.int32),
        pltpu.VMEM((CHUNK,), jnp.int32),
        pltpu.VMEM((2 * CHUNK,), jnp.float32),
        pltpu.VMEM((NW, L), jnp.int32),
        pltpu.VMEM((CHUNK, D), jnp.float32),
        pltpu.VMEM((2, CHUNK // 4, D), jnp.int32),
        pltpu.VMEM((4, CHUNK // 4), jnp.int32),
        pltpu.SemaphoreType.DMA,
        pltpu.SemaphoreType.DMA,
        pltpu.SemaphoreType.DMA,
        pltpu.SemaphoreType.DMA,
    ],
    compiler_params=pltpu.CompilerParams(needs_layout_passes=False,
                                         use_tc_tiling_on_sc=False),
)
def _partition_kernel(fav_hbm, ret_hbm, wfl_hbm, emb_hbm, cnt_hbm,
                      packed_out, emb_out,
                      fav_v, ret_v, w_v, cnt_v, emb_v, packed_v, dest_v,
                      ldsem, stsem, pksem0, pksem1):
    w = _wid()
    base = w * CHUNK
    emb_cp = pltpu.async_copy(emb_hbm.at[pl.ds(base, CHUNK), :], emb_v, ldsem)
    pltpu.sync_copy(fav_hbm.at[pl.ds(base, CHUNK)], fav_v)
    pltpu.sync_copy(ret_hbm.at[pl.ds(base, CHUNK)], ret_v)
    pltpu.sync_copy(wfl_hbm.at[pl.ds(2 * base, 2 * CHUNK)], w_v)
    pltpu.sync_copy(cnt_hbm, cnt_v)

    widx = lax.iota(jnp.int32, L)
    zero = jnp.zeros((L,), jnp.int32)
    c0 = plsc.load_gather(cnt_v, [widx, zero])
    c1 = plsc.load_gather(cnt_v, [widx + L, zero])
    npos = jnp.sum(c0) + jnp.sum(c1)
    pos_base = (jnp.sum(jnp.where(widx < w, c0, 0))
                + jnp.sum(jnp.where(widx + L < w, c1, 0)))
    neg_base = npos + base - pos_base

    npos_s = jnp.full((L,), npos, jnp.int32).astype(jnp.float32)
    den_s = jnp.float32(BS) - npos_s
    nw_s = jnp.where(den_s == 0.0, jnp.float32(0.0),
                     (jnp.float32(2 * BS) - npos_s) / den_s)
    pos_base_s = jnp.full((L,), pos_base, jnp.int32)
    neg_base_s = jnp.full((L,), neg_base, jnp.int32)
    lane = lax.iota(jnp.int32, L)
    ones = jnp.ones((L,), jnp.float32)
    cols = [jnp.full((L,), c, jnp.int32) for c in range(4)]
    carry_p = jnp.zeros((L,), jnp.int32)
    emb_cp_waited = False
    pk_copies = []
    emb_copies = []
    SUB = CHUNK // 4
    for sub in range(4):
        buf = packed_v.at[sub % 2]
        if sub >= 2:
            pk_copies[sub - 2].wait()
        for k in range(SUB // L):
            b = sub * (SUB // L) + k
            f = fav_v[pl.ds(b * L, L)]
            r = ret_v[pl.ds(b * L, L)]
            m = (f == 1) | (r == 1)
            mi = m.astype(jnp.int32)
            excl = plsc.cumsum(mi) - mi
            posd = pos_base_s + carry_p + excl
            negd = neg_base_s + (b * L - carry_p) + (lane - excl)
            dest = jnp.where(m, posd, negd)
            dest = jnp.where(dest >= BS, BS + (dest & (PAD - 1)), dest)
            dest_v[sub, pl.ds(k * L, L)] = dest
            carry_p = carry_p + plsc.all_reduce_population_count(m)
            rows = lane + (b * L)
            w0 = plsc.load_gather(w_v, [rows * 2])
            w1 = plsc.load_gather(w_v, [rows * 2 + 1])
            factor = jnp.where(m, ones, nw_s)
            lrows = lane + k * L
            vals = (f, r, plsc.bitcast(w0 * factor, jnp.int32),
                    plsc.bitcast(w1 * factor, jnp.int32))
            for c in range(4):
                plsc.store_scatter(buf, [lrows, cols[c]], vals[c])
        if not emb_cp_waited:
            emb_cp.wait()
            emb_cp_waited = True
        idx = dest_v.at[sub]
        emb_copies.append(
            pltpu.async_copy(emb_v.at[pl.ds(sub * SUB, SUB), :],
                             emb_out.at[idx], stsem))
        pk_copies.append(pltpu.async_copy(buf, packed_out.at[idx],
                                          (pksem0, pksem1)[sub % 2]))
    for c in emb_copies:
        c.wait()
    for sub in (2, 3):
        pk_copies[sub].wait()


def kernel(fav, retweet, embedding, weights):
    fav1 = fav.reshape(B)
    ret1 = retweet.reshape(B)
    wfl = weights.reshape(2 * B)
    cnts = _count_kernel(fav1, ret1)
    packed, embp = _partition_kernel(fav1, ret1, wfl, embedding, cnts)
    out_fav = packed[:BS, 0:1]
    out_ret = packed[:BS, 1:2]
    out_w = lax.bitcast_convert_type(packed[:BS, 2:4], jnp.float32)
    return out_fav, out_ret, embp[:BS], out_w
